# baseline (speedup 1.0000x reference)
import functools

import jax
import jax.numpy as jnp
from jax import lax
from jax.experimental import pallas as pl
from jax.experimental.pallas import tpu as pltpu
from jax.experimental.pallas import tpu_sc as plsc


def _make_sc_gather(v, d, b):
    nc, ns = 2, 16
    nw = nc * ns
    b_per_w = b // nw
    chunk = b_per_w // 2
    mesh = plsc.VectorSubcoreMesh(core_axis_name="c", subcore_axis_name="s")

    @functools.partial(
        pl.kernel, mesh=mesh,
        out_type=jax.ShapeDtypeStruct((b, d), jnp.float32),
        scratch_types=[
            pltpu.VMEM((b_per_w,), jnp.int32),
            pltpu.VMEM((b_per_w, d), jnp.float32),
            pltpu.SemaphoreType.DMA,
        ],
    )
    def k(table_hbm, idx_hbm, out_hbm, idx_v, rows_v, sem):
        wid = lax.axis_index("s") * nc + lax.axis_index("c")
        base = wid * b_per_w
        pltpu.sync_copy(idx_hbm.at[pl.ds(base, b_per_w)], idx_v)
        for j in range(2):
            pltpu.async_copy(
                table_hbm.at[idx_v.at[pl.ds(j * chunk, chunk)]],
                rows_v.at[pl.ds(j * chunk, chunk)], sem).wait()
        pltpu.sync_copy(rows_v, out_hbm.at[pl.ds(base, b_per_w)])

    return k


def _mm_nn(a, b):
    return jax.lax.dot_general(a, b, (((1,), (0,)), ((), ())),
                               preferred_element_type=jnp.float32)


def _mm_nt(a, b):
    return jax.lax.dot_general(a, b, (((1,), (1,)), ((), ())),
                               preferred_element_type=jnp.float32)


def _onehot_bf16(idx_col, n):
    cols = jax.lax.broadcasted_iota(jnp.int32, (idx_col.shape[0], n), 1)
    return jnp.where(cols == idx_col, 1.0, 0.0).astype(jnp.bfloat16)


def _group_matrix(f):
    r = jax.lax.broadcasted_iota(jnp.int32, (7 * f, f), 0)
    c = jax.lax.broadcasted_iota(jnp.int32, (7 * f, f), 1)
    u = (r - c * 7).astype(jnp.uint32)
    return jnp.where(u < 7, 1.0, 0.0).astype(jnp.float32)


def _gather7(h, es, m):
    n, f = h.shape
    hi = h.astype(jnp.bfloat16)
    lo = (h - hi.astype(jnp.float32)).astype(jnp.bfloat16)
    hcat = jnp.concatenate([hi, lo], axis=1)
    blocks = []
    for d in range(6):
        g = _mm_nn(es[d][0:m, :], hcat)
        blocks.append(g[:, 0:f] + g[:, f:2 * f])
    blocks.append(h[0:m, :])
    return jnp.concatenate(blocks, axis=1)


def _bn_lrelu(h, g, be):
    mu = jnp.mean(h, axis=0, keepdims=True)
    c = h - mu
    var = jnp.mean(c * c, axis=0, keepdims=True)
    y = c * jax.lax.rsqrt(var + 1e-5) * g + be
    return jnp.where(y >= 0.0, y, 0.2 * y)


def _row(ref):
    return ref[...].reshape(1, -1)


def _body(mat1_ref, w1_ref, b1_ref, g1_ref, be1_ref, w2_hbm, b2_ref, g2_ref,
          be2_ref, w3_hbm, b3_ref, g3_ref, be3_ref, wfc_ref, bfc_ref,
          no_ref, out_ref, w2_v, w3_v, sem2, sem3):
    cp2 = pltpu.make_async_copy(w2_hbm, w2_v, sem2)
    cp3 = pltpu.make_async_copy(w3_hbm, w3_v, sem3)
    cp2.start()
    cp3.start()

    no = no_ref[...]
    no1 = no[0:642, :]
    no2 = no[642:804, :]
    no3 = no[804:846, :]

    es1 = [_onehot_bf16(no1[0:162, d:d + 1], 642) for d in range(6)]
    h = _mm_nt(mat1_ref[...], w1_ref[...]) + _row(b1_ref)
    h = _bn_lrelu(h, _row(g1_ref), _row(be1_ref))
    h = _mm_nn(_gather7(h, es1, 162), _group_matrix(128)) * (1.0 / 7.0)

    cp2.wait()
    es2 = [_onehot_bf16(no2[:, d:d + 1], 162) for d in range(6)]
    h = _mm_nt(_gather7(h, es2, 162), w2_v[...]) + _row(b2_ref)
    h = _bn_lrelu(h, _row(g2_ref), _row(be2_ref))
    h = _mm_nn(_gather7(h, es2, 42), _group_matrix(256)) * (1.0 / 7.0)

    cp3.wait()
    es3 = [_onehot_bf16(no3[:, d:d + 1], 42) for d in range(6)]
    h = _mm_nt(_gather7(h, es3, 42), w3_v[...]) + _row(b3_ref)
    h = _bn_lrelu(h, _row(g3_ref), _row(be3_ref))
    h = _mm_nn(_gather7(h, es3, 12), _group_matrix(512)) * (1.0 / 7.0)

    t = h * wfc_ref[...]
    s = jnp.sum(t) * (1.0 / 12.0) + bfc_ref[0]
    sv = jnp.full((1, 1), s, jnp.float32)
    out_ref[...] = 1.0 / (1.0 + jnp.exp(-sv))


def kernel(x, W1, b1, g1, be1, W2, b2, g2, be2, W3, b3, g3, be3, Wfc, bfc,
           no642, no162, no42):
    noall = jnp.concatenate([no642.astype(jnp.int32), no162.astype(jnp.int32),
                             no42.astype(jnp.int32)]).reshape(846, 7)
    idx_pad = jnp.concatenate([no642.astype(jnp.int32),
                               jnp.zeros((114,), jnp.int32)])
    x_pad = jnp.pad(x, ((0, 0), (0, 96)))
    gath = _make_sc_gather(642, 128, 4608)(x_pad, idx_pad)
    mat1 = gath[0:4494, 0:32].reshape(642, 224)
    vspec = pl.BlockSpec(memory_space=pltpu.VMEM)
    aspec = pl.BlockSpec(memory_space=pltpu.HBM)
    sspec = pl.BlockSpec(memory_space=pltpu.SMEM)
    out = pl.pallas_call(
        _body,
        out_shape=jax.ShapeDtypeStruct((1, 1), jnp.float32),
        in_specs=[vspec, vspec, vspec, vspec, vspec,
                  aspec, vspec, vspec, vspec,
                  aspec, vspec, vspec, vspec,
                  vspec, sspec, vspec],
        out_specs=vspec,
        scratch_shapes=[
            pltpu.VMEM((256, 896), jnp.float32),
            pltpu.VMEM((512, 1792), jnp.float32),
            pltpu.SemaphoreType.DMA,
            pltpu.SemaphoreType.DMA,
        ],
    )(mat1, W1, b1, g1, be1, W2, b2, g2, be2, W3, b3, g3, be3,
      Wfc, bfc.reshape(1), noall)
    return out.reshape(1)


# speedup vs baseline: 2.5897x; 2.5897x over previous
import jax
import jax.numpy as jnp
from jax.experimental import pallas as pl
from jax.experimental.pallas import tpu as pltpu


def _mm_nn(a, b):
    return jax.lax.dot_general(a, b, (((1,), (0,)), ((), ())),
                               preferred_element_type=jnp.float32)


def _mm_nt(a, b):
    return jax.lax.dot_general(a, b, (((1,), (1,)), ((), ())),
                               preferred_element_type=jnp.float32)


def _onehot_bf16(idx_col, n):
    cols = jax.lax.broadcasted_iota(jnp.int32, (idx_col.shape[0], n), 1)
    return jnp.where(cols == idx_col, 1.0, 0.0).astype(jnp.bfloat16)


def _group_matrix(f):
    r = jax.lax.broadcasted_iota(jnp.int32, (7 * f, f), 0)
    c = jax.lax.broadcasted_iota(jnp.int32, (7 * f, f), 1)
    u = (r - c * 7).astype(jnp.uint32)
    return jnp.where(u < 7, 1.0, 0.0).astype(jnp.float32)


def _gather7(h, es, m):
    n, f = h.shape
    hi = h.astype(jnp.bfloat16)
    lo = (h - hi.astype(jnp.float32)).astype(jnp.bfloat16)
    hcat = jnp.concatenate([hi, lo], axis=1)
    blocks = []
    for d in range(6):
        g = _mm_nn(es[d][0:m, :], hcat)
        blocks.append(g[:, 0:f] + g[:, f:2 * f])
    blocks.append(h[0:m, :])
    return jnp.concatenate(blocks, axis=1)


def _bn_lrelu(h, g, be):
    mu = jnp.mean(h, axis=0, keepdims=True)
    c = h - mu
    var = jnp.mean(c * c, axis=0, keepdims=True)
    y = c * jax.lax.rsqrt(var + 1e-5) * g + be
    return jnp.where(y >= 0.0, y, 0.2 * y)


def _row(ref):
    return ref[...].reshape(1, -1)


def _body(x_ref, w1_ref, b1_hbm, g1_hbm, be1_hbm, w2_hbm, b2_hbm, g2_hbm,
          be2_hbm, w3_hbm, b3_hbm, g3_hbm, be3_hbm, wfc_ref, bfc_ref,
          no_ref, out_ref, w2_v, w3_v, sv1, sv2, sv3, sem2, sem3, semv):
    cp2 = pltpu.make_async_copy(w2_hbm, w2_v, sem2)
    cp3 = pltpu.make_async_copy(w3_hbm, w3_v, sem3)
    cp2.start()
    cp3.start()
    cv1 = [pltpu.make_async_copy(r, sv1.at[pl.ds(i, 1), :], semv)
           for i, r in enumerate((b1_hbm, g1_hbm, be1_hbm))]
    cv2 = [pltpu.make_async_copy(r, sv2.at[pl.ds(i, 1), :], semv)
           for i, r in enumerate((b2_hbm, g2_hbm, be2_hbm))]
    cv3 = [pltpu.make_async_copy(r, sv3.at[pl.ds(i, 1), :], semv)
           for i, r in enumerate((b3_hbm, g3_hbm, be3_hbm))]
    for c in cv1 + cv2 + cv3:
        c.start()

    x = x_ref[...]
    no = no_ref[...]
    no1 = no[0:642, :]
    no2 = no[642:804, :]
    no3 = no[804:846, :]

    es1 = [_onehot_bf16(no1[:, d:d + 1], 642) for d in range(6)]
    mat = _gather7(x, es1, 642)
    for c in cv1:
        c.wait()
    h = _mm_nt(mat, w1_ref[...]) + sv1[0:1, :]
    h = _bn_lrelu(h, sv1[1:2, :], sv1[2:3, :])
    h = _mm_nn(_gather7(h, es1, 162), _group_matrix(128)) * (1.0 / 7.0)

    cp2.wait()
    es2 = [_onehot_bf16(no2[:, d:d + 1], 162) for d in range(6)]
    mat = _gather7(h, es2, 162)
    for c in cv2:
        c.wait()
    h = _mm_nt(mat, w2_v[...]) + sv2[0:1, :]
    h = _bn_lrelu(h, sv2[1:2, :], sv2[2:3, :])
    h = _mm_nn(_gather7(h, es2, 42), _group_matrix(256)) * (1.0 / 7.0)

    cp3.wait()
    es3 = [_onehot_bf16(no3[:, d:d + 1], 42) for d in range(6)]
    mat = _gather7(h, es3, 42)
    for c in cv3:
        c.wait()
    h = _mm_nt(mat, w3_v[...]) + sv3[0:1, :]
    h = _bn_lrelu(h, sv3[1:2, :], sv3[2:3, :])
    h = _mm_nn(_gather7(h, es3, 12), _group_matrix(512)) * (1.0 / 7.0)

    t = h * wfc_ref[...]
    s = jnp.sum(t) * (1.0 / 12.0) + bfc_ref[0]
    sv = jnp.full((1, 1), s, jnp.float32)
    out_ref[...] = 1.0 / (1.0 + jnp.exp(-sv))


def kernel(x, W1, b1, g1, be1, W2, b2, g2, be2, W3, b3, g3, be3, Wfc, bfc,
           no642, no162, no42):
    noall = jnp.concatenate([no642.astype(jnp.int32), no162.astype(jnp.int32),
                             no42.astype(jnp.int32)]).reshape(846, 7)
    vspec = pl.BlockSpec(memory_space=pltpu.VMEM)
    aspec = pl.BlockSpec(memory_space=pltpu.HBM)
    sspec = pl.BlockSpec(memory_space=pltpu.SMEM)
    out = pl.pallas_call(
        _body,
        out_shape=jax.ShapeDtypeStruct((1, 1), jnp.float32),
        in_specs=[vspec, vspec, aspec, aspec, aspec,
                  aspec, aspec, aspec, aspec,
                  aspec, aspec, aspec, aspec,
                  vspec, sspec, vspec],
        out_specs=vspec,
        scratch_shapes=[
            pltpu.VMEM((256, 896), jnp.float32),
            pltpu.VMEM((512, 1792), jnp.float32),
            pltpu.VMEM((3, 128), jnp.float32),
            pltpu.VMEM((3, 256), jnp.float32),
            pltpu.VMEM((3, 512), jnp.float32),
            pltpu.SemaphoreType.DMA,
            pltpu.SemaphoreType.DMA,
            pltpu.SemaphoreType.DMA,
        ],
    )(x, W1, b1.reshape(1, -1), g1.reshape(1, -1), be1.reshape(1, -1),
      W2, b2.reshape(1, -1), g2.reshape(1, -1), be2.reshape(1, -1),
      W3, b3.reshape(1, -1), g3.reshape(1, -1), be3.reshape(1, -1),
      Wfc, bfc.reshape(1), noall)
    return out.reshape(1)


# speedup vs baseline: 2.6782x; 1.0342x over previous
import jax
import jax.numpy as jnp
from jax.experimental import pallas as pl
from jax.experimental.pallas import tpu as pltpu


def _mm_nn(a, b):
    return jax.lax.dot_general(a, b, (((1,), (0,)), ((), ())),
                               preferred_element_type=jnp.float32)


def _mm_nt(a, b):
    return jax.lax.dot_general(a, b, (((1,), (1,)), ((), ())),
                               preferred_element_type=jnp.float32)


def _onehot_bf16(idx_col, n):
    cols = jax.lax.broadcasted_iota(jnp.int32, (idx_col.shape[0], n), 1)
    return jnp.where(cols == idx_col, 1.0, 0.0).astype(jnp.bfloat16)


def _group_matrix(f):
    r = jax.lax.broadcasted_iota(jnp.int32, (7 * f, f), 0)
    c = jax.lax.broadcasted_iota(jnp.int32, (7 * f, f), 1)
    u = (r - c * 7).astype(jnp.uint32)
    return jnp.where(u < 7, 1.0, 0.0).astype(jnp.float32)


def _gather7(h, es, m):
    n, f = h.shape
    hi = h.astype(jnp.bfloat16)
    lo = (h - hi.astype(jnp.float32)).astype(jnp.bfloat16)
    hcat = jnp.concatenate([hi, lo], axis=1)
    blocks = []
    for d in range(6):
        g = _mm_nn(es[d][0:m, :], hcat)
        blocks.append(g[:, 0:f] + g[:, f:2 * f])
    blocks.append(h[0:m, :])
    return jnp.concatenate(blocks, axis=1)


def _bn_lrelu(h, g, be):
    mu = jnp.mean(h, axis=0, keepdims=True)
    c = h - mu
    var = jnp.mean(c * c, axis=0, keepdims=True)
    y = c * jax.lax.rsqrt(var + 1e-5) * g + be
    return jnp.where(y >= 0.0, y, 0.2 * y)


def _row(ref):
    return ref[...].reshape(1, -1)


def _body(x_ref, w1_ref, b1_ref, g1_ref, be1_ref, w2_hbm, b2_ref, g2_ref,
          be2_ref, w3_hbm, b3_ref, g3_ref, be3_ref, wfc_ref, bfc_ref,
          no_ref, out_ref, w2_v, w3_v, sem2a, sem2b, sem3a, sem3b, sem3c,
          sem3d):
    cp2 = [pltpu.make_async_copy(w2_hbm.at[pl.ds(o, 128), :],
                                 w2_v.at[pl.ds(o, 128), :], s)
           for o, s in ((0, sem2a), (128, sem2b))]
    cp3 = [pltpu.make_async_copy(w3_hbm.at[pl.ds(o, 128), :],
                                 w3_v.at[pl.ds(o, 128), :], s)
           for o, s in ((0, sem3a), (128, sem3b), (256, sem3c), (384, sem3d))]
    for c in cp2 + cp3:
        c.start()

    x = x_ref[...]
    no = no_ref[...]
    no1 = no[0:642, :]
    no2 = no[642:804, :]
    no3 = no[804:846, :]

    es1 = [_onehot_bf16(no1[:, d:d + 1], 642) for d in range(6)]
    h = _mm_nt(_gather7(x, es1, 642), w1_ref[...]) + _row(b1_ref)
    h = _bn_lrelu(h, _row(g1_ref), _row(be1_ref))
    h = _mm_nn(_gather7(h, es1, 162), _group_matrix(128)) * (1.0 / 7.0)

    for c in cp2:
        c.wait()
    es2 = [_onehot_bf16(no2[:, d:d + 1], 162) for d in range(6)]
    h = _mm_nt(_gather7(h, es2, 162), w2_v[...]) + _row(b2_ref)
    h = _bn_lrelu(h, _row(g2_ref), _row(be2_ref))
    h = _mm_nn(_gather7(h, es2, 42), _group_matrix(256)) * (1.0 / 7.0)

    for c in cp3:
        c.wait()
    es3 = [_onehot_bf16(no3[:, d:d + 1], 42) for d in range(6)]
    h = _mm_nt(_gather7(h, es3, 42), w3_v[...]) + _row(b3_ref)
    h = _bn_lrelu(h, _row(g3_ref), _row(be3_ref))
    h = _mm_nn(_gather7(h, es3, 12), _group_matrix(512)) * (1.0 / 7.0)

    t = h * wfc_ref[...]
    s = jnp.sum(t) * (1.0 / 12.0) + bfc_ref[0]
    sv = jnp.full((1, 1), s, jnp.float32)
    out_ref[...] = 1.0 / (1.0 + jnp.exp(-sv))


def kernel(x, W1, b1, g1, be1, W2, b2, g2, be2, W3, b3, g3, be3, Wfc, bfc,
           no642, no162, no42):
    noall = jnp.concatenate([no642.astype(jnp.int32), no162.astype(jnp.int32),
                             no42.astype(jnp.int32)]).reshape(846, 7)
    vspec = pl.BlockSpec(memory_space=pltpu.VMEM)
    aspec = pl.BlockSpec(memory_space=pltpu.HBM)
    sspec = pl.BlockSpec(memory_space=pltpu.SMEM)
    out = pl.pallas_call(
        _body,
        out_shape=jax.ShapeDtypeStruct((1, 1), jnp.float32),
        in_specs=[vspec, vspec, vspec, vspec, vspec,
                  aspec, vspec, vspec, vspec,
                  aspec, vspec, vspec, vspec,
                  vspec, sspec, vspec],
        out_specs=vspec,
        scratch_shapes=[
            pltpu.VMEM((256, 896), jnp.float32),
            pltpu.VMEM((512, 1792), jnp.float32),
            pltpu.SemaphoreType.DMA,
            pltpu.SemaphoreType.DMA,
            pltpu.SemaphoreType.DMA,
            pltpu.SemaphoreType.DMA,
            pltpu.SemaphoreType.DMA,
            pltpu.SemaphoreType.DMA,
        ],
    )(x, W1, b1, g1, be1, W2, b2, g2, be2, W3, b3, g3, be3,
      Wfc, bfc.reshape(1), noall)
    return out.reshape(1)


# speedup vs baseline: 2.6967x; 1.0069x over previous
import jax
import jax.numpy as jnp
from jax.experimental import pallas as pl
from jax.experimental.pallas import tpu as pltpu


def _mm_nn(a, b):
    return jax.lax.dot_general(a, b, (((1,), (0,)), ((), ())),
                               preferred_element_type=jnp.float32)


def _mm_nt(a, b):
    return jax.lax.dot_general(a, b, (((1,), (1,)), ((), ())),
                               preferred_element_type=jnp.float32)


def _onehot_bf16(idx_col, n):
    cols = jax.lax.broadcasted_iota(jnp.int32, (idx_col.shape[0], n), 1)
    return jnp.where(cols == idx_col, 1.0, 0.0).astype(jnp.bfloat16)


def _group_matrix(f):
    r = jax.lax.broadcasted_iota(jnp.int32, (7 * f, f), 0)
    c = jax.lax.broadcasted_iota(jnp.int32, (7 * f, f), 1)
    u = (r - c * 7).astype(jnp.uint32)
    return jnp.where(u < 7, 1.0, 0.0).astype(jnp.float32)


def _gather7(h, es, m):
    n, f = h.shape
    hi = h.astype(jnp.bfloat16)
    lo = (h - hi.astype(jnp.float32)).astype(jnp.bfloat16)
    hcat = jnp.concatenate([hi, lo], axis=1)
    blocks = []
    for d in range(6):
        g = _mm_nn(es[d][0:m, :], hcat)
        blocks.append(g[:, 0:f] + g[:, f:2 * f])
    blocks.append(h[0:m, :])
    return jnp.concatenate(blocks, axis=1)


def _bn_lrelu(h, g, be):
    mu = jnp.mean(h, axis=0, keepdims=True)
    c = h - mu
    var = jnp.mean(c * c, axis=0, keepdims=True)
    y = c * jax.lax.rsqrt(var + 1e-5) * g + be
    return jnp.where(y >= 0.0, y, 0.2 * y)


def _row(ref):
    return ref[...].reshape(1, -1)


def _body(x_ref, w1_ref, b1_ref, g1_ref, be1_ref, w2_hbm, b2_ref, g2_ref,
          be2_ref, w3_hbm, b3_ref, g3_ref, be3_ref, wfc_ref, bfc_ref,
          no_ref, out_ref, w2_v, w3_v, sem2, sem3):
    cp2 = pltpu.make_async_copy(w2_hbm, w2_v, sem2)
    cp3 = pltpu.make_async_copy(w3_hbm, w3_v, sem3)
    cp2.start()
    cp3.start()

    x = x_ref[...]
    no = no_ref[...]
    no1 = no[0:642, :]
    no2 = no[642:804, :]
    no3 = no[804:846, :]

    es1 = [_onehot_bf16(no1[:, d:d + 1], 642) for d in range(6)]
    es2 = [_onehot_bf16(no2[:, d:d + 1], 162) for d in range(6)]
    es3 = [_onehot_bf16(no3[:, d:d + 1], 42) for d in range(6)]

    h = _mm_nt(_gather7(x, es1, 642), w1_ref[...]) + _row(b1_ref)
    h = _bn_lrelu(h, _row(g1_ref), _row(be1_ref))
    h = _mm_nn(_gather7(h, es1, 162), _group_matrix(128)) * (1.0 / 7.0)

    cp2.wait()
    h = _mm_nt(_gather7(h, es2, 162), w2_v[...]) + _row(b2_ref)
    h = _bn_lrelu(h, _row(g2_ref), _row(be2_ref))
    h = _mm_nn(_gather7(h, es2, 42), _group_matrix(256)) * (1.0 / 7.0)

    cp3.wait()
    h = _mm_nt(_gather7(h, es3, 42), w3_v[...]) + _row(b3_ref)
    h = _bn_lrelu(h, _row(g3_ref), _row(be3_ref))
    h = _mm_nn(_gather7(h, es3, 12), _group_matrix(512)) * (1.0 / 7.0)

    t = h * wfc_ref[...]
    s = jnp.sum(t) * (1.0 / 12.0) + bfc_ref[0]
    sv = jnp.full((1, 1), s, jnp.float32)
    out_ref[...] = 1.0 / (1.0 + jnp.exp(-sv))


def kernel(x, W1, b1, g1, be1, W2, b2, g2, be2, W3, b3, g3, be3, Wfc, bfc,
           no642, no162, no42):
    noall = jnp.concatenate([no642.astype(jnp.int32), no162.astype(jnp.int32),
                             no42.astype(jnp.int32)]).reshape(846, 7)
    vspec = pl.BlockSpec(memory_space=pltpu.VMEM)
    aspec = pl.BlockSpec(memory_space=pltpu.HBM)
    sspec = pl.BlockSpec(memory_space=pltpu.SMEM)
    out = pl.pallas_call(
        _body,
        out_shape=jax.ShapeDtypeStruct((1, 1), jnp.float32),
        in_specs=[vspec, vspec, vspec, vspec, vspec,
                  aspec, vspec, vspec, vspec,
                  aspec, vspec, vspec, vspec,
                  vspec, sspec, vspec],
        out_specs=vspec,
        scratch_shapes=[
            pltpu.VMEM((256, 896), jnp.float32),
            pltpu.VMEM((512, 1792), jnp.float32),
            pltpu.SemaphoreType.DMA,
            pltpu.SemaphoreType.DMA,
        ],
    )(x, W1, b1, g1, be1, W2, b2, g2, be2, W3, b3, g3, be3,
      Wfc, bfc.reshape(1), noall)
    return out.reshape(1)
